# Initial kernel scaffold; baseline (speedup 1.0000x reference)
import jax
import jax.numpy as jnp
from jax.experimental import pallas as pl


def kernel(x, g1, g2, Wq, Wdkv, Wuk, Wuv, Wo, Wiq, Wik, w_head, Wr, W1, W2):
    raise NotImplementedError("write your pallas kernel here")


# speedup vs baseline: 2.4742x; 2.4742x over previous
import jax
import jax.numpy as jnp
import numpy as np
from jax.experimental import pallas as pl
from jax.experimental.pallas import tpu as pltpu

B, S, D = 1, 2048, 1024
H, DH = 16, 64
KVR = 512
IH, ID = 4, 64
SPARSE_TOP_K = 512
E, TOPK, F = 8, 2, 2048
EPS = 1e-6

BR = 256


def _rms(x, g):
    return x * jax.lax.rsqrt(jnp.mean(x * x, axis=-1, keepdims=True) + EPS) * g


def _proj_kernel(x_ref, g1_ref, Wq_ref, Wdkv_ref, Wuk_ref, Wuv_ref, Wiq_ref,
                 Wik_ref, q_ref, k_ref, v_ref, iq_ref, ik_ref):
    x = x_ref[...]
    h = _rms(x, g1_ref[...])
    q_ref[...] = jnp.dot(h, Wq_ref[...], preferred_element_type=jnp.float32)
    ckv = jnp.dot(h, Wdkv_ref[...], preferred_element_type=jnp.float32)
    k_ref[...] = jnp.dot(ckv, Wuk_ref[...], preferred_element_type=jnp.float32)
    v_ref[...] = jnp.dot(ckv, Wuv_ref[...], preferred_element_type=jnp.float32)
    iq_ref[...] = jnp.dot(h, Wiq_ref[...], preferred_element_type=jnp.float32)
    ik_ref[...] = jnp.dot(h, Wik_ref[...], preferred_element_type=jnp.float32)


def _dotT(a, b):
    return jax.lax.dot_general(a, b, (((1,), (1,)), ((), ())),
                               preferred_element_type=jnp.float32)


def _attn_kernel(x_ref, q_ref, iq_ref, k_ref, v_ref, ik_ref, wh_ref, Wo_ref,
                 g2_ref, Wr_ref, isc_ref, x1_ref, h2_ref, gates_ref,
                 fsum_ref, psum_ref):
    i = pl.program_id(0)
    qstart = i * BR

    ik = ik_ref[...]
    scores = jnp.zeros((BR, S), jnp.float32)
    for hh in range(IH):
        raw = _dotT(iq_ref[:, hh * ID:(hh + 1) * ID], ik)
        scores = scores + wh_ref[0, hh] * jnp.maximum(raw, 0.0)
    isc_ref[...] = scores

    rows = qstart + jax.lax.broadcasted_iota(jnp.int32, (BR, S), 0)
    cols = jax.lax.broadcasted_iota(jnp.int32, (BR, S), 1)
    valid = cols <= rows

    bits = jnp.where(valid, jax.lax.bitcast_convert_type(scores, jnp.int32), -1)
    prefix = jnp.zeros((BR, 1), jnp.int32)
    for b in range(30, -1, -1):
        cand = prefix | (1 << b)
        cnt = jnp.sum((bits >= cand).astype(jnp.int32), axis=1, keepdims=True)
        prefix = jnp.where(cnt >= SPARSE_TOP_K, cand, prefix)
    nvalid = rows[:, :1] + 1
    allowed = valid & ((nvalid <= SPARSE_TOP_K) | (bits >= prefix))

    scale = np.float32(1.0 / np.sqrt(DH))
    neg = jnp.float32(-1e9)
    k = k_ref[...]
    v = v_ref[...]
    q = q_ref[...]
    outs = []
    for hh in range(H):
        sl = slice(hh * DH, (hh + 1) * DH)
        att = _dotT(q[:, sl], k[:, sl]) * scale
        att = jnp.where(allowed, att, neg)
        m = jnp.max(att, axis=1, keepdims=True)
        p = jnp.exp(att - m)
        p = p / jnp.sum(p, axis=1, keepdims=True)
        outs.append(jnp.dot(p, v[:, sl], preferred_element_type=jnp.float32))
    attn = jnp.concatenate(outs, axis=1)

    x1 = x_ref[...] + jnp.dot(attn, Wo_ref[...],
                              preferred_element_type=jnp.float32)
    x1_ref[...] = x1
    h2 = _rms(x1, g2_ref[...])
    h2_ref[...] = h2

    logits = jnp.dot(h2, Wr_ref[...], preferred_element_type=jnp.float32)
    lm = jnp.max(logits, axis=1, keepdims=True)
    pe = jnp.exp(logits - lm)
    probs = pe / jnp.sum(pe, axis=1, keepdims=True)
    eidx = jax.lax.broadcasted_iota(jnp.int32, (BR, E), 1)
    m1 = jnp.max(probs, axis=1, keepdims=True)
    i1 = jnp.min(jnp.where(probs == m1, eidx, E), axis=1, keepdims=True)
    pr2 = jnp.where(eidx == i1, -1.0, probs)
    m2 = jnp.max(pr2, axis=1, keepdims=True)
    i2 = jnp.min(jnp.where(pr2 == m2, eidx, E), axis=1, keepdims=True)
    tsum = m1 + m2
    is1 = eidx == i1
    is2 = eidx == i2
    gates_ref[...] = (jnp.where(is1, m1 / tsum, 0.0) +
                      jnp.where(is2, m2 / tsum, 0.0))
    sel = is1.astype(jnp.float32) + is2.astype(jnp.float32)

    @pl.when(i == 0)
    def _():
        fsum_ref[...] = jnp.zeros_like(fsum_ref)
        psum_ref[...] = jnp.zeros_like(psum_ref)

    fsum_ref[...] += jnp.sum(sel, axis=0, keepdims=True)
    psum_ref[...] += jnp.sum(probs, axis=0, keepdims=True)


def _moe_kernel(x1_ref, h2_ref, gates_ref, W1_ref, W2_ref, out_ref):
    e = pl.program_id(1)

    @pl.when(e == 0)
    def _():
        out_ref[...] = x1_ref[...]

    eidx = jax.lax.broadcasted_iota(jnp.int32, (BR, E), 1)
    g = jnp.sum(jnp.where(eidx == e, gates_ref[...], 0.0), axis=1,
                keepdims=True)
    mid = jax.nn.gelu(jnp.dot(h2_ref[...], W1_ref[...].reshape(D, F),
                              preferred_element_type=jnp.float32))
    out_ref[...] += g * jnp.dot(mid, W2_ref[...].reshape(F, D),
                                preferred_element_type=jnp.float32)


def kernel(x, g1, g2, Wq, Wdkv, Wuk, Wuv, Wo, Wiq, Wik, w_head, Wr, W1, W2):
    x2d = x.reshape(S, D)
    g1r = g1.reshape(1, D)
    g2r = g2.reshape(1, D)
    whr = w_head.reshape(1, IH)

    nblk = S // BR
    full = lambda shape: pl.BlockSpec(shape, lambda i: tuple(0 for _ in shape))
    rowblk = pl.BlockSpec((BR, None), lambda i: (i, 0))

    q, k, v, iq, ik = pl.pallas_call(
        _proj_kernel,
        grid=(nblk,),
        in_specs=[
            pl.BlockSpec((BR, D), lambda i: (i, 0)),
            full((1, D)),
            full((D, H * DH)),
            full((D, KVR)),
            full((KVR, H * DH)),
            full((KVR, H * DH)),
            full((D, IH * ID)),
            full((D, ID)),
        ],
        out_specs=[
            pl.BlockSpec((BR, H * DH), lambda i: (i, 0)),
            pl.BlockSpec((BR, H * DH), lambda i: (i, 0)),
            pl.BlockSpec((BR, H * DH), lambda i: (i, 0)),
            pl.BlockSpec((BR, IH * ID), lambda i: (i, 0)),
            pl.BlockSpec((BR, ID), lambda i: (i, 0)),
        ],
        out_shape=[
            jax.ShapeDtypeStruct((S, H * DH), jnp.float32),
            jax.ShapeDtypeStruct((S, H * DH), jnp.float32),
            jax.ShapeDtypeStruct((S, H * DH), jnp.float32),
            jax.ShapeDtypeStruct((S, IH * ID), jnp.float32),
            jax.ShapeDtypeStruct((S, ID), jnp.float32),
        ],
        compiler_params=pltpu.CompilerParams(
            dimension_semantics=("arbitrary",)),
    )(x2d, g1r, Wq, Wdkv, Wuk, Wuv, Wiq, Wik)

    isc, x1, h2, gates, fsum, psum = pl.pallas_call(
        _attn_kernel,
        grid=(nblk,),
        in_specs=[
            pl.BlockSpec((BR, D), lambda i: (i, 0)),
            pl.BlockSpec((BR, H * DH), lambda i: (i, 0)),
            pl.BlockSpec((BR, IH * ID), lambda i: (i, 0)),
            full((S, H * DH)),
            full((S, H * DH)),
            full((S, ID)),
            full((1, IH)),
            full((H * DH, D)),
            full((1, D)),
            full((D, E)),
        ],
        out_specs=[
            pl.BlockSpec((BR, S), lambda i: (i, 0)),
            pl.BlockSpec((BR, D), lambda i: (i, 0)),
            pl.BlockSpec((BR, D), lambda i: (i, 0)),
            pl.BlockSpec((BR, E), lambda i: (i, 0)),
            pl.BlockSpec((1, E), lambda i: (0, 0)),
            pl.BlockSpec((1, E), lambda i: (0, 0)),
        ],
        out_shape=[
            jax.ShapeDtypeStruct((S, S), jnp.float32),
            jax.ShapeDtypeStruct((S, D), jnp.float32),
            jax.ShapeDtypeStruct((S, D), jnp.float32),
            jax.ShapeDtypeStruct((S, E), jnp.float32),
            jax.ShapeDtypeStruct((1, E), jnp.float32),
            jax.ShapeDtypeStruct((1, E), jnp.float32),
        ],
        compiler_params=pltpu.CompilerParams(
            dimension_semantics=("arbitrary",)),
    )(x2d, q, iq, k, v, ik, whr, Wo, g2r, Wr)

    out = pl.pallas_call(
        _moe_kernel,
        grid=(nblk, E),
        in_specs=[
            pl.BlockSpec((BR, D), lambda i, e: (i, 0)),
            pl.BlockSpec((BR, D), lambda i, e: (i, 0)),
            pl.BlockSpec((BR, E), lambda i, e: (i, 0)),
            pl.BlockSpec((1, D, F), lambda i, e: (e, 0, 0)),
            pl.BlockSpec((1, F, D), lambda i, e: (e, 0, 0)),
        ],
        out_specs=pl.BlockSpec((BR, D), lambda i, e: (i, 0)),
        out_shape=jax.ShapeDtypeStruct((S, D), jnp.float32),
        compiler_params=pltpu.CompilerParams(
            dimension_semantics=("arbitrary", "arbitrary")),
    )(x1, h2, gates, W1, W2)

    f_frac = fsum[0] / np.float32(B * S)
    p_mean = psum[0] / np.float32(B * S)
    aux_loss = E * jnp.sum(f_frac * p_mean)
    return out.reshape(B, S, D), aux_loss, isc.reshape(B, S, S)
